# Initial kernel scaffold; baseline (speedup 1.0000x reference)
import jax
import jax.numpy as jnp
from jax.experimental import pallas as pl


def kernel(inputs, projections, control_points, component_weights):
    raise NotImplementedError("write your pallas kernel here")


# speedup vs baseline: 12.1142x; 12.1142x over previous
import jax
import jax.numpy as jnp
from jax.experimental import pallas as pl
from jax.experimental.pallas import tpu as pltpu

_NCOMP = 3
_GRID = 6
_ODIM = 128
_TN = 1024


def _fused_body(x_ref, p_ref, w_ref, g_ref, t_ref, o_ref):
    x = x_ref[...]
    proj = jnp.dot(x, p_ref[...], preferred_element_type=jnp.float32)
    proj = jnp.clip(proj, -0.99, 0.99)
    g = g_ref[...]
    cs = []
    for i in range(_NCOMP):
        p_i = proj[:, i : i + 1]
        c = jnp.maximum(1.0 - 2.5 * jnp.abs(p_i - g), 0.0)
        cs.append(c * w_ref[0, i])
    coef = jnp.concatenate(cs, axis=1)
    out = jnp.dot(coef, t_ref[...], preferred_element_type=jnp.float32)
    o_ref[...] = jnp.tanh(out)


def kernel(inputs, projections, control_points, component_weights):
    B, S, D = inputs.shape
    N = B * S
    x = inputs.reshape(N, D)
    pmat = projections[:, :, 0].T
    w = component_weights.reshape(1, _NCOMP)
    grid = jnp.linspace(-1.0, 1.0, _GRID).reshape(1, _GRID).astype(jnp.float32)
    tbl = control_points.reshape(_NCOMP * _GRID, _ODIM)

    out = pl.pallas_call(
        _fused_body,
        grid=(N // _TN,),
        in_specs=[
            pl.BlockSpec((_TN, D), lambda i: (i, 0)),
            pl.BlockSpec((D, _NCOMP), lambda i: (0, 0)),
            pl.BlockSpec((1, _NCOMP), lambda i: (0, 0)),
            pl.BlockSpec((1, _GRID), lambda i: (0, 0)),
            pl.BlockSpec((_NCOMP * _GRID, _ODIM), lambda i: (0, 0)),
        ],
        out_specs=pl.BlockSpec((_TN, _ODIM), lambda i: (i, 0)),
        out_shape=jax.ShapeDtypeStruct((N, _ODIM), jnp.float32),
        compiler_params=pltpu.CompilerParams(
            dimension_semantics=("arbitrary",)
        ),
    )(x, pmat, w, grid, tbl)
    return out.reshape(B, S, _ODIM)
